# Initial kernel scaffold; baseline (speedup 1.0000x reference)
import jax
import jax.numpy as jnp
from jax.experimental import pallas as pl


def kernel(x, edge_index, Ws1, Wd1, as1, ad1, b1, Wl1, bl1, Ws2, Wd2, as2, ad2, b2, Wl2, bl2):
    raise NotImplementedError("write your pallas kernel here")


# speedup vs baseline: 18.6219x; 18.6219x over previous
import functools

import jax
import jax.numpy as jnp
from jax import lax
from jax.experimental import pallas as pl
from jax.experimental.pallas import tpu as pltpu
from jax.experimental.pallas import tpu_sc as plsc

N = 10000
E = 320000
D = 128
H = 128

NPAD = 10240
BLK = 512
GRID = NPAD // BLK

NW = 32
CHUNK = 128
CPW = 79
EPW = CPW * CHUNK
EPAD = NW * EPW
RPW = NPAD // 16


def _prologue_body(x_ref, ws_ref, wd_ref, atts_ref, attd_ref,
                   xs_ref, asad_ref, mx_ref):
    i = pl.program_id(0)
    xb = x_ref[...]
    xs = jnp.dot(xb, ws_ref[...], preferred_element_type=jnp.float32)
    xs_ref[...] = xs
    a_s = jnp.dot(xs, atts_ref[...], preferred_element_type=jnp.float32)
    wdat = jnp.dot(wd_ref[...], attd_ref[...],
                   preferred_element_type=jnp.float32)
    a_d = jnp.dot(xb, wdat, preferred_element_type=jnp.float32)
    asad_ref[...] = jnp.concatenate([a_s, a_d], axis=1)
    bm = jnp.max(a_s)

    @pl.when(i == 0)
    def _():
        for j in range(16):
            mx_ref[j] = bm

    @pl.when(i > 0)
    def _():
        for j in range(16):
            mx_ref[j] = jnp.maximum(mx_ref[j], bm)


def _prologue(xp, Ws, Wd, atts, attd):
    return pl.pallas_call(
        _prologue_body,
        grid=(GRID,),
        in_specs=[
            pl.BlockSpec((BLK, D), lambda i: (i, 0)),
            pl.BlockSpec((D, H), lambda i: (0, 0)),
            pl.BlockSpec((D, H), lambda i: (0, 0)),
            pl.BlockSpec((H, 1), lambda i: (0, 0)),
            pl.BlockSpec((H, 1), lambda i: (0, 0)),
        ],
        out_specs=[
            pl.BlockSpec((BLK, H), lambda i: (i, 0)),
            pl.BlockSpec((BLK, 2), lambda i: (i, 0)),
            pl.BlockSpec(memory_space=pltpu.SMEM),
        ],
        out_shape=[
            jax.ShapeDtypeStruct((NPAD, H), jnp.float32),
            jax.ShapeDtypeStruct((NPAD, 2), jnp.float32),
            jax.ShapeDtypeStruct((16,), jnp.float32),
        ],
    )(xp, Ws, Wd, atts.reshape(H, 1), attd.reshape(H, 1))


def _epilogue_body(acc_ref, s_ref, x_ref, wl_ref, b_ref, bl_ref, o_ref,
                   *, relu):
    acc = acc_ref[0] + acc_ref[1]
    s = jnp.sum(s_ref[...], axis=0)
    gat = acc / (s[:, None] + 1e-16) + b_ref[...]
    lin = jnp.dot(x_ref[...], wl_ref[...],
                  preferred_element_type=jnp.float32) + bl_ref[...]
    r = gat + lin
    o_ref[...] = jnp.maximum(r, 0.0) if relu else r


def _epilogue(acc, svals, xp, Wl, b, bl, relu):
    return pl.pallas_call(
        functools.partial(_epilogue_body, relu=relu),
        grid=(GRID,),
        in_specs=[
            pl.BlockSpec((2, BLK, H), lambda i: (0, i, 0)),
            pl.BlockSpec((2, BLK), lambda i: (0, i)),
            pl.BlockSpec((BLK, D), lambda i: (i, 0)),
            pl.BlockSpec((D, H), lambda i: (0, 0)),
            pl.BlockSpec((1, H), lambda i: (0, 0)),
            pl.BlockSpec((1, H), lambda i: (0, 0)),
        ],
        out_specs=pl.BlockSpec((BLK, H), lambda i: (i, 0)),
        out_shape=jax.ShapeDtypeStruct((NPAD, H), jnp.float32),
    )(acc, svals, xp, Wl, b.reshape(1, H), bl.reshape(1, H))


def _sc_edge_body(src_hbm, dst_hbm, asad_hbm, mx_hbm, xs_hbm,
                  zrows_hbm, zvec_hbm, acc_out, s_out,
                  asad_v, src_c, dst_c, val_c, rows_v, mx_v,
                  sem, acc_sh, s_sh):
    cid = lax.axis_index("c")
    sid = lax.axis_index("s")
    wid = sid * 2 + cid

    pltpu.sync_copy(asad_hbm, asad_v)
    pltpu.sync_copy(mx_hbm, mx_v)
    pltpu.sync_copy(zrows_hbm.at[pl.ds(sid * RPW, RPW)],
                    acc_sh.at[pl.ds(sid * RPW, RPW)])

    @pl.when(sid == 0)
    def _():
        pltpu.sync_copy(zvec_hbm, s_sh)

    plsc.subcore_barrier()

    mxa = mx_v[...]

    @pl.loop(0, CPW)
    def _chunk(c):
        pltpu.sync_copy(src_hbm.at[wid].at[c], src_c)
        pltpu.sync_copy(dst_hbm.at[wid].at[c], dst_c)
        for i in range(8):
            s_idx = src_c[pl.ds(i * 16, 16)]
            d_idx = dst_c[pl.ds(i * 16, 16)]
            a_s = plsc.load_gather(asad_v, [s_idx * 2])
            a_d = plsc.load_gather(asad_v, [d_idx * 2 + 1])
            t = a_s + a_d
            t = jnp.where(t >= 0.0, t, 0.2 * t)
            m = mxa + a_d
            m = jnp.where(m >= 0.0, m, 0.2 * m)
            val_c[pl.ds(i * 16, 16)] = jnp.exp(t - m)
        pltpu.sync_copy(val_c, s_sh.at[dst_c], add=True)
        pltpu.async_copy(xs_hbm.at[src_c], rows_v, sem).wait()

        @pl.loop(0, CHUNK)
        def _scale(j):
            vb = plsc.load_gather(val_c, [jnp.full((16,), j, jnp.int32)])
            for k in range(8):
                sl = (j, pl.ds(k * 16, 16))
                rows_v[sl] = rows_v[sl] * vb

        pltpu.sync_copy(rows_v, acc_sh.at[dst_c], add=True)

    plsc.subcore_barrier()

    pltpu.sync_copy(acc_sh.at[pl.ds(sid * RPW, RPW)],
                    acc_out.at[cid].at[pl.ds(sid * RPW, RPW)])

    @pl.when(sid == 0)
    def _():
        pltpu.sync_copy(s_sh, s_out.at[cid])


def _sc_edge(src3, dst3, asad_flat, mx, xs, zrows, zvec):
    mesh = plsc.VectorSubcoreMesh(core_axis_name="c", subcore_axis_name="s")
    f = pl.kernel(
        _sc_edge_body,
        out_type=[
            jax.ShapeDtypeStruct((2, NPAD, H), jnp.float32),
            jax.ShapeDtypeStruct((2, NPAD), jnp.float32),
        ],
        mesh=mesh,
        compiler_params=pltpu.CompilerParams(needs_layout_passes=False),
        scratch_types=[
            pltpu.VMEM((2 * NPAD,), jnp.float32),
            pltpu.VMEM((CHUNK,), jnp.int32),
            pltpu.VMEM((CHUNK,), jnp.int32),
            pltpu.VMEM((CHUNK,), jnp.float32),
            pltpu.VMEM((CHUNK, H), jnp.float32),
            pltpu.VMEM((16,), jnp.float32),
            pltpu.SemaphoreType.DMA,
            pltpu.VMEM_SHARED((NPAD, H), jnp.float32),
            pltpu.VMEM_SHARED((NPAD,), jnp.float32),
        ],
    )
    return f(src3, dst3, asad_flat, mx, xs, zrows, zvec)


def kernel(x, edge_index, Ws1, Wd1, as1, ad1, b1, Wl1, bl1,
           Ws2, Wd2, as2, ad2, b2, Wl2, bl2):
    src = edge_index[0].astype(jnp.int32)
    dst = edge_index[1].astype(jnp.int32)
    pad = EPAD - E
    src3 = jnp.concatenate([src, jnp.full((pad,), N, jnp.int32)]
                           ).reshape(NW, CPW, CHUNK)
    dst3 = jnp.concatenate([dst, jnp.full((pad,), N, jnp.int32)]
                           ).reshape(NW, CPW, CHUNK)
    xp = jnp.pad(x, ((0, NPAD - N), (0, 0)))
    zrows = jnp.zeros((NPAD, H), jnp.float32)
    zvec = jnp.zeros((NPAD,), jnp.float32)

    def layer(xin, Ws, Wd, atts, attd, b, Wl, bl, relu):
        xs, asad, mx = _prologue(xin, Ws, Wd, atts, attd)
        acc, svals = _sc_edge(src3, dst3, asad.reshape(2 * NPAD), mx,
                              xs, zrows, zvec)
        return _epilogue(acc, svals, xin, Wl, b, bl, relu)

    h = layer(xp, Ws1, Wd1, as1, ad1, b1, Wl1, bl1, True)
    out = layer(h, Ws2, Wd2, as2, ad2, b2, Wl2, bl2, False)
    return out[:N]
